# baseline (speedup 1.0000x reference)
import jax
import jax.numpy as jnp
from jax.experimental import pallas as pl
from jax.experimental.pallas import tpu as pltpu

H = 8
D = 64
M = 8192
B = 1024
K = 64
IN = 512

_TM = 512
_TMD = 1024


def _mask_body(x_ref, wq_ref, bq_ref, bp_ref, mask_ref):
    x = x_ref[0]
    q = jnp.sum(wq_ref[:] * x[None, None, :], axis=2) + bq_ref[:]
    qn = q * jax.lax.rsqrt(jnp.sum(q * q, axis=1, keepdims=True))
    s = jnp.sum(bp_ref[:] * qn[:, None, :], axis=2)

    i = jax.lax.bitcast_convert_type(s, jnp.int32)
    key = jnp.where(i < 0, i ^ jnp.int32(0x7FFFFFFF), i)

    def step(it, cur):
        bit = 31 - it
        cand = cur + (jnp.int32(1) << bit)
        cnt = jnp.sum((key >= cand).astype(jnp.float32), axis=1, keepdims=True)
        return jnp.where(cnt >= K, cand, cur)

    kth = jax.lax.fori_loop(0, 32, step, jnp.full((H, 1), jnp.int32(-(2**31))))

    gt = (key > kth).astype(jnp.float32)
    n_gt = jnp.sum(gt, axis=1, keepdims=True)
    tie = (key == kth).astype(jnp.float32)
    r = jax.lax.broadcasted_iota(jnp.int32, (B, B), 0)
    c = jax.lax.broadcasted_iota(jnp.int32, (B, B), 1)
    lt = (r < c).astype(jnp.float32)
    tie_rank = jnp.dot(tie, lt, preferred_element_type=jnp.float32)
    sel_tie = tie * (tie_rank < (K - n_gt)).astype(jnp.float32)
    mask_ref[:] = gt + sel_tie


def _attn_body(mask_ref, w_ref, attn_ref):
    w = w_ref[:]
    m = mask_ref[:]
    attn_ref[:] = jnp.sum(w * m[:, None, :], axis=2) * (1.0 / K)


def _softmax_body(attn_ref, p_ref):
    a = attn_ref[:]
    s = (2.0 * a) / (1.0 + a)
    l = s * 10.0
    l = l - jnp.max(l, axis=1, keepdims=True)
    e = jnp.exp(l)
    p_ref[:] = e / jnp.sum(e, axis=1, keepdims=True)


def _retrieve_body(p_ref, mem_ref, out_ref):
    t = pl.program_id(0)
    mem = mem_ref[:]
    inv = jax.lax.rsqrt(jnp.sum(mem * mem, axis=2))
    w = p_ref[:] * inv
    o = jnp.sum(mem * w[:, :, None], axis=1)

    @pl.when(t == 0)
    def _():
        out_ref[:] = jnp.zeros_like(out_ref)

    out_ref[:] += o


def kernel(x, Wq, bq, bin_proj, weight_matrix, memories):
    x2 = x.reshape(1, IN)
    wq3 = Wq.reshape(H, D, IN)
    bq2 = bq.reshape(H, D)

    mask = pl.pallas_call(
        _mask_body,
        out_shape=jax.ShapeDtypeStruct((H, B), jnp.float32),
    )(x2, wq3, bq2, bin_proj)

    attn = pl.pallas_call(
        _attn_body,
        grid=(M // _TM,),
        in_specs=[
            pl.BlockSpec((H, B), lambda t: (0, 0)),
            pl.BlockSpec((H, _TM, B), lambda t: (0, t, 0)),
        ],
        out_specs=pl.BlockSpec((H, _TM), lambda t: (0, t)),
        out_shape=jax.ShapeDtypeStruct((H, M), jnp.float32),
    )(mask, weight_matrix)

    p = pl.pallas_call(
        _softmax_body,
        out_shape=jax.ShapeDtypeStruct((H, M), jnp.float32),
    )(attn)

    out = pl.pallas_call(
        _retrieve_body,
        grid=(M // _TMD,),
        in_specs=[
            pl.BlockSpec((H, _TMD), lambda t: (0, t)),
            pl.BlockSpec((H, _TMD, D), lambda t: (0, t, 0)),
        ],
        out_specs=pl.BlockSpec((H, D), lambda t: (0, 0)),
        out_shape=jax.ShapeDtypeStruct((H, D), jnp.float32),
    )(p, memories)

    return (out * jnp.sqrt(float(D))).reshape(H * D)
